# Initial kernel scaffold; baseline (speedup 1.0000x reference)
import jax
import jax.numpy as jnp
from jax.experimental import pallas as pl


def kernel(queries_0, queries_1, queries_2, keys_0, keys_1, keys_2, values_0, values_1, values_2):
    raise NotImplementedError("write your pallas kernel here")


# speedup vs baseline: 81.6783x; 81.6783x over previous
import functools

import jax
import jax.numpy as jnp
from jax import lax
from jax.experimental import pallas as pl
from jax.experimental.pallas import tpu as pltpu
from jax.experimental.pallas import tpu_sc as plsc

_NH = 8
_D = 16
_K = 8
_LB = 64


def _topk8(vals, cid, bound):
    w = vals
    hit = jnp.zeros(vals.shape, jnp.bool_)
    scores, ids = [], []
    red_axes = tuple(range(1, vals.ndim))
    for _ in range(_K):
        m = jnp.max(w, axis=red_axes, keepdims=True)
        sel = w == m
        idxj = jnp.min(jnp.where(sel, cid, bound), axis=red_axes, keepdims=True)
        hitj = cid == idxj
        w = jnp.where(hitj, -1.0, w)
        hit = jnp.logical_or(hit, hitj)
        scores.append(m.reshape(vals.shape[0]))
        ids.append(idxj.reshape(vals.shape[0]))
    return scores, ids, hit


def _coarse_body(q_ref, k_ref, v_ref, msg_ref, sc_ref, tix_ref):
    h = pl.program_id(0)
    q = q_ref[0]
    k = k_ref[0]
    v = v_ref[0]
    qk = lax.dot_general(q.astype(jnp.bfloat16), k.astype(jnp.bfloat16),
                         (((1,), (1,)), ((), ())),
                         preferred_element_type=jnp.float32) * 0.25
    m0 = jnp.max(qk, axis=1, keepdims=True)
    e = jnp.exp(qk - m0)
    p = e / jnp.sum(e, axis=1, keepdims=True)
    s_n = p.shape[1]
    iota = lax.broadcasted_iota(jnp.int32, p.shape, 1)
    scores, ids, hit = _topk8(p, iota, s_n)
    pm = jnp.where(hit, 0.0, p)
    msg_ref[0] = lax.dot_general(pm.astype(jnp.bfloat16),
                                 v.astype(jnp.bfloat16),
                                 (((1,), (0,)), ((), ())),
                                 preferred_element_type=jnp.float32)
    sc_ref[0] = jnp.concatenate([s[:, None] for s in scores], axis=1)
    tix_ref[0] = (jnp.concatenate([i[:, None] for i in ids], axis=1)
                  + h * 1024)


def _fine_body(last, w_grid, q_ref, kv_ref, sc_ref, tix_ref, mp_ref,
               *out_refs):
    h = pl.program_id(0)
    q4 = q_ref[0]
    kv = kv_ref[0]
    kg = kv[:, :, 0:4, :]
    vg = kv[:, :, 4:8, :]
    sc_prev = sc_ref[0]
    tix_prev = tix_ref[0] - h * ((w_grid // 2) ** 2)
    mp = mp_ref[0]
    lb = q4.shape[0]
    cid4 = (lax.broadcasted_iota(jnp.int32, (lb, _K, 4), 1) * 4
            + lax.broadcasted_iota(jnp.int32, (lb, _K, 4), 2))
    if not last:
        lane = lax.broadcasted_iota(jnp.int32, (lb, 32), 1)
        kk = lane // 4
        ff = lane % 4
        pos = jnp.zeros((lb, 32), jnp.int32)
        for k in range(_K):
            pos = jnp.where(kk == k, tix_prev[:, k][:, None], pos)
        r, c = pos // (w_grid // 2), pos % (w_grid // 2)
        x, y = ff // 2, ff % 2
        cand_pos = (2 * r + x) * w_grid + (2 * c + y)
    msgs, sc_outs, tix_outs = [], [], []
    for w in range(4):
        qw = q4[:, w, :].astype(jnp.bfloat16).astype(jnp.float32)
        kgb = kg.astype(jnp.bfloat16).astype(jnp.float32)
        z = jnp.sum(qw[:, None, None, :] * kgb, axis=-1) * 0.25
        m0 = jnp.max(z, axis=-1, keepdims=True)
        e = jnp.exp(z - m0)
        p = e / jnp.sum(e, axis=-1, keepdims=True)
        a = p * sc_prev[:, :, None]
        if last:
            am = a
        else:
            scores, ids, hit = _topk8(a, cid4, 32)
            am = jnp.where(hit, 0.0, a)
        amb = am.astype(jnp.bfloat16).astype(jnp.float32)
        vgb = vg.astype(jnp.bfloat16).astype(jnp.float32)
        mw = jnp.sum(jnp.sum(amb[:, :, :, None] * vgb, axis=2), axis=1)
        msgs.append((mp + mw)[:, None, :])
        if not last:
            sc_outs.append(
                jnp.concatenate([s[:, None] for s in scores], axis=1)[:, None, :])
            pjs = []
            for j in range(_K):
                cidj = ids[j][:, None]
                pjs.append(jnp.sum(jnp.where(lane == cidj, cand_pos, 0),
                                   axis=1, keepdims=True))
            tix_outs.append((jnp.concatenate(pjs, axis=1)
                             + h * (w_grid * w_grid))[:, None, :])
    out_refs[0][0] = jnp.concatenate(msgs, axis=1)
    if not last:
        out_refs[1][0] = jnp.concatenate(sc_outs, axis=1)
        out_refs[2][0] = jnp.concatenate(tix_outs, axis=1)


def _coarse_call(q, k, v):
    s_n = q.shape[1]
    nlb = s_n // _LB
    return pl.pallas_call(
        _coarse_body,
        grid=(_NH, nlb),
        in_specs=[
            pl.BlockSpec((1, _LB, _D), lambda h, l: (h, l, 0)),
            pl.BlockSpec((1, s_n, _D), lambda h, l: (h, 0, 0)),
            pl.BlockSpec((1, s_n, _D), lambda h, l: (h, 0, 0)),
        ],
        out_specs=[
            pl.BlockSpec((1, _LB, _D), lambda h, l: (h, l, 0)),
            pl.BlockSpec((1, _LB, _K), lambda h, l: (h, l, 0)),
            pl.BlockSpec((1, _LB, _K), lambda h, l: (h, l, 0)),
        ],
        out_shape=[
            jax.ShapeDtypeStruct((_NH, s_n, _D), jnp.float32),
            jax.ShapeDtypeStruct((_NH, s_n, _K), jnp.float32),
            jax.ShapeDtypeStruct((_NH, s_n, _K), jnp.int32),
        ],
    )(q, k, v)


def _fine_call(last, w_grid, q4, kv, sc_prev, tix, mp):
    lp = q4.shape[1]
    nlb = lp // _LB
    out_specs = [pl.BlockSpec((1, _LB, 4, _D), lambda h, l: (h, l, 0, 0))]
    out_shape = [jax.ShapeDtypeStruct((_NH, lp, 4, _D), jnp.float32)]
    if not last:
        out_specs += [
            pl.BlockSpec((1, _LB, 4, _K), lambda h, l: (h, l, 0, 0)),
            pl.BlockSpec((1, _LB, 4, _K), lambda h, l: (h, l, 0, 0)),
        ]
        out_shape += [
            jax.ShapeDtypeStruct((_NH, lp, 4, _K), jnp.float32),
            jax.ShapeDtypeStruct((_NH, lp, 4, _K), jnp.int32),
        ]
    return pl.pallas_call(
        functools.partial(_fine_body, last, w_grid),
        grid=(_NH, nlb),
        in_specs=[
            pl.BlockSpec((1, _LB, 4, _D), lambda h, l: (h, l, 0, 0)),
            pl.BlockSpec((1, _LB, _K, 8, _D), lambda h, l: (h, l, 0, 0, 0)),
            pl.BlockSpec((1, _LB, _K), lambda h, l: (h, l, 0)),
            pl.BlockSpec((1, _LB, _K), lambda h, l: (h, l, 0)),
            pl.BlockSpec((1, _LB, _D), lambda h, l: (h, l, 0)),
        ],
        out_specs=out_specs,
        out_shape=out_shape,
    )(q4, kv, sc_prev, tix, mp)


def _sc_gather(kvtab, idx):
    b_n = idx.shape[0]
    info = plsc.get_sparse_core_info()
    nc, ns = info.num_cores, info.num_subcores
    nw = nc * ns
    b_per_w = b_n // nw
    ch = 512
    nch = b_per_w // ch
    mesh = plsc.VectorSubcoreMesh(core_axis_name="c", subcore_axis_name="s")

    @functools.partial(
        pl.kernel, mesh=mesh,
        out_type=jax.ShapeDtypeStruct((b_n, 8 * _D), jnp.float32),
        scratch_types=[
            pltpu.VMEM((ch,), jnp.int32),
            pltpu.VMEM((ch, 8 * _D), jnp.float32),
            pltpu.SemaphoreType.DMA,
        ],
    )
    def gk(tab_hbm, idx_hbm, out_hbm, idx_v, rows_v, sem):
        wid = lax.axis_index("s") * nc + lax.axis_index("c")
        base = wid * b_per_w
        for j in range(nch):
            off = base + j * ch
            pltpu.sync_copy(idx_hbm.at[pl.ds(off, ch)], idx_v)
            pltpu.async_copy(tab_hbm.at[idx_v], rows_v, sem).wait()
            pltpu.sync_copy(rows_v, out_hbm.at[pl.ds(off, ch)])

    return gk(kvtab, idx)


def _to_heads(x):
    hh, ww = x.shape[2], x.shape[3]
    t = jnp.transpose(x[0], (1, 2, 0)).reshape(hh * ww, _NH, _D)
    return jnp.transpose(t, (1, 0, 2))


def _to_kv_table(kx, vx):
    hh, ww = kx.shape[2], kx.shape[3]

    def blk(x):
        t = x[0].reshape(_NH, _D, hh // 2, 2, ww // 2, 2)
        t = jnp.transpose(t, (0, 2, 4, 3, 5, 1))
        return t.reshape(_NH, (hh // 2) * (ww // 2), 4, _D)

    t = jnp.concatenate([blk(kx), blk(vx)], axis=2)
    return t.reshape(_NH * (hh // 2) * (ww // 2), 8 * _D)


def _to_quad_q(x):
    c, hh, ww = x.shape[1], x.shape[2], x.shape[3]
    t = x[0].reshape(c, hh // 2, 2, ww // 2, 2)
    t = jnp.transpose(t, (1, 3, 2, 4, 0)).reshape(
        (hh // 2) * (ww // 2), 4, _NH, _D)
    return jnp.transpose(t, (2, 0, 1, 3))


def _quad_deinterleave(x, hw_half):
    tail = x.shape[3:]
    t = x.reshape((_NH, hw_half, hw_half, 2, 2) + tail)
    t = jnp.transpose(t, (0, 1, 3, 2, 4) + tuple(range(5, 5 + len(tail))))
    return t.reshape((_NH, 4 * hw_half * hw_half) + tail)


def kernel(queries_0, queries_1, queries_2, keys_0, keys_1, keys_2,
           values_0, values_1, values_2):
    q2 = _to_heads(queries_2)
    k2 = _to_heads(keys_2)
    v2 = _to_heads(values_2)
    msg0, sc0, tix0 = _coarse_call(q2, k2, v2)

    kv1f = _sc_gather(_to_kv_table(keys_1, values_1), tix0.reshape(-1))
    kv1 = kv1f.reshape(_NH, 1024, _K, 8, _D)
    q1 = _to_quad_q(queries_1)
    msg1, sc1, tix1 = _fine_call(False, 64, q1, kv1, sc0, tix0, msg0)
    msg1 = _quad_deinterleave(msg1, 32)
    sc1 = _quad_deinterleave(sc1, 32)
    tix1 = _quad_deinterleave(tix1, 32)

    kv2f = _sc_gather(_to_kv_table(keys_0, values_0), tix1.reshape(-1))
    kv2 = kv2f.reshape(_NH, 4096, _K, 8, _D)
    q0 = _to_quad_q(queries_0)
    (msg2,) = _fine_call(True, 128, q0, kv2, sc1, tix1, msg1)
    msg2 = _quad_deinterleave(msg2, 64)
    return jnp.transpose(msg2, (1, 0, 2))[None]
